# baseline (speedup 1.0000x reference)
import functools

import jax
import jax.numpy as jnp
from jax import lax
from jax.experimental import pallas as pl
from jax.experimental.pallas import tpu as pltpu
from jax.experimental.pallas import tpu_sc as plsc

VOCAB = 1000000
D = 32
B = 16384
L = 200

_NC = 2
_NS = 16
_NW = _NC * _NS
_VTILES = VOCAB // 128
_NT = _VTILES + 1
_LB = L // 8
_BB = B // 128
_PAIRS = _LB * _BB
_PER_W = _PAIRS // _NW
_NTILES_OUT = L * (D // 8) * _BB

_MESH = dict(core_axis_name="c", subcore_axis_name="s")


def _make_detile():
    @functools.partial(
        pl.kernel,
        mesh=plsc.VectorSubcoreMesh(**_MESH),
        compiler_params=pltpu.CompilerParams(
            use_tc_tiling_on_sc=True, needs_layout_passes=False
        ),
        out_type=jax.ShapeDtypeStruct((VOCAB * D // 128, 128), jnp.float32),
        scratch_types=[
            pltpu.VMEM((D, 128), jnp.float32),
            pltpu.VMEM((D, 128), jnp.float32),
        ],
    )
    def detile(wt_hbm, out2_hbm, slab_v, outb_v):
        wid = lax.axis_index("s") * _NC + lax.axis_index("c")
        iota = lax.iota(jnp.int32, 16)
        rows_g = [(16 * g + iota) // 4 for g in range(8)]
        cols_g = [((16 * g + iota) % 4) * D for g in range(8)]

        def tile_body(k, carry):
            t = wid + k * _NW

            @pl.when(t < _NT)
            def _():
                pltpu.sync_copy(wt_hbm.at[:, pl.ds(t * 128, 128)], slab_v)
                for d in range(D):
                    for g in range(8):
                        v = slab_v[d, pl.ds(g * 16, 16)]
                        plsc.store_scatter(
                            outb_v, [rows_g[g], cols_g[g] + d], v
                        )

                @pl.when(t < _VTILES)
                def _():
                    pltpu.sync_copy(outb_v, out2_hbm.at[pl.ds(t * 32, 32), :])

                @pl.when(t == _VTILES)
                def _():
                    pltpu.sync_copy(
                        outb_v.at[pl.ds(0, 16), :],
                        out2_hbm.at[pl.ds(t * 32, 16), :],
                    )

            return carry

        lax.fori_loop(0, (_NT + _NW - 1) // _NW, tile_body, 0)

    return detile


def _make_gather():
    @functools.partial(
        pl.kernel,
        mesh=plsc.VectorSubcoreMesh(**_MESH),
        compiler_params=pltpu.CompilerParams(
            use_tc_tiling_on_sc=False, needs_layout_passes=False
        ),
        out_type=jax.ShapeDtypeStruct((_NTILES_OUT, 8, 128), jnp.float32),
        scratch_types=[
            pltpu.VMEM((8, 128), jnp.int32),
            pltpu.VMEM((128, D), jnp.float32),
            pltpu.VMEM((D, 128), jnp.float32),
            pltpu.SemaphoreType.DMA,
        ],
    )
    def gat(tbl_hbm, xt_hbm, o_hbm, idx_v, rows_v, outt_v, sem):
        wid = lax.axis_index("s") * _NC + lax.axis_index("c")
        iota = lax.iota(jnp.int32, 16)
        rows_t = [16 * g + iota for g in range(2)]

        def pair_body(k, carry):
            p = wid + k * _NW
            lb = p // _BB
            bb = p % _BB
            pltpu.sync_copy(xt_hbm.at[lb, bb], idx_v)

            def li_body(q, carry2):
                pltpu.async_copy(tbl_hbm.at[idx_v.at[q]], rows_v, sem).wait()
                for b in range(128):
                    for g in range(2):
                        v = rows_v[b, pl.ds(g * 16, 16)]
                        plsc.store_scatter(
                            outt_v, [rows_t[g], iota * 0 + b], v
                        )
                row0 = ((lb * 8 + q) * 4) * _BB + bb
                for db in range(4):
                    pltpu.sync_copy(
                        outt_v.at[pl.ds(8 * db, 8), :],
                        o_hbm.at[row0 + db * _BB],
                    )
                return carry2

            lax.fori_loop(0, 8, li_body, 0)
            return carry

        lax.fori_loop(0, _PER_W, pair_body, 0)

    return gat


def _make_retile():
    @functools.partial(
        pl.kernel,
        mesh=plsc.VectorSubcoreMesh(**_MESH),
        compiler_params=pltpu.CompilerParams(
            use_tc_tiling_on_sc=True, needs_layout_passes=False
        ),
        out_type=jax.ShapeDtypeStruct((L, D, B), jnp.float32),
    )
    def retile(t_hbm, o_hbm):
        wid = lax.axis_index("s") * _NC + lax.axis_index("c")

        def tile_body(k, carry):
            r = wid + k * _NW
            lbb = r // _BB
            bb = r % _BB
            l = lbb // 4
            db = lbb % 4
            pltpu.sync_copy(
                t_hbm.at[r],
                o_hbm.at[l, pl.ds(8 * db, 8), pl.ds(bb * 128, 128)],
            )
            return carry

        lax.fori_loop(0, _NTILES_OUT // _NW, tile_body, 0)

    return retile


_detile = _make_detile()
_gather = _make_gather()
_retile = _make_retile()


def kernel(x, weight):
    wt = weight.T
    tbl4 = _detile(wt)
    tbl = tbl4.reshape(VOCAB, D)
    xq = x.T.reshape(_LB, 8, _BB, 128).transpose(0, 2, 1, 3)
    t = _gather(tbl, xq)
    o = _retile(t)
    return o.transpose(2, 0, 1)


# speedup vs baseline: 4.2827x; 4.2827x over previous
import functools

import jax
import jax.numpy as jnp
from jax import lax
from jax.experimental import pallas as pl
from jax.experimental.pallas import tpu as pltpu
from jax.experimental.pallas import tpu_sc as plsc

VOCAB = 1000000
D = 32
B = 16384
L = 200

_NC = 2
_NS = 16
_NW = _NC * _NS
_VTILES = VOCAB // 128
_NT = _VTILES + 1
_LB = L // 8
_BB = B // 128
_PAIRS = _LB * _BB
_PER_W = _PAIRS // _NW
_NTILES_OUT = L * (D // 8) * _BB

_MESH = dict(core_axis_name="c", subcore_axis_name="s")


def _make_detile():
    @functools.partial(
        pl.kernel,
        mesh=plsc.VectorSubcoreMesh(**_MESH),
        compiler_params=pltpu.CompilerParams(
            use_tc_tiling_on_sc=True, needs_layout_passes=False
        ),
        out_type=jax.ShapeDtypeStruct((VOCAB * D // 128, 128), jnp.float32),
        scratch_types=[
            pltpu.VMEM((D, 128), jnp.float32),
            pltpu.VMEM((D, 128), jnp.float32),
        ],
    )
    def detile(wt_hbm, out2_hbm, slab_v, outb_v):
        wid = lax.axis_index("s") * _NC + lax.axis_index("c")
        iota = lax.iota(jnp.int32, 16)
        rows_g = [(16 * g + iota) // 4 for g in range(8)]
        cols_g = [((16 * g + iota) % 4) * D for g in range(8)]

        def tile_body(k, carry):
            t = wid + k * _NW

            @pl.when(t < _NT)
            def _():
                pltpu.sync_copy(wt_hbm.at[:, pl.ds(t * 128, 128)], slab_v)
                for d in range(D):
                    for g in range(8):
                        v = slab_v[d, pl.ds(g * 16, 16)]
                        plsc.store_scatter(
                            outb_v, [rows_g[g], cols_g[g] + d], v
                        )

                @pl.when(t < _VTILES)
                def _():
                    pltpu.sync_copy(outb_v, out2_hbm.at[pl.ds(t * 32, 32), :])

                @pl.when(t == _VTILES)
                def _():
                    pltpu.sync_copy(
                        outb_v.at[pl.ds(0, 16), :],
                        out2_hbm.at[pl.ds(t * 32, 16), :],
                    )

            return carry

        lax.fori_loop(0, (_NT + _NW - 1) // _NW, tile_body, 0)

    return detile


def _make_gather():
    @functools.partial(
        pl.kernel,
        mesh=plsc.VectorSubcoreMesh(**_MESH),
        compiler_params=pltpu.CompilerParams(
            use_tc_tiling_on_sc=False, needs_layout_passes=False
        ),
        out_type=jax.ShapeDtypeStruct((_NTILES_OUT, 8, 128), jnp.float32),
        scratch_types=[
            pltpu.VMEM((8, 128), jnp.int32),
            pltpu.VMEM((128, D), jnp.float32),
            pltpu.VMEM((D, 128), jnp.float32),
            pltpu.SemaphoreType.DMA,
        ],
    )
    def gat(tbl_hbm, xt_hbm, o_hbm, idx_v, rows_v, outt_v, sem):
        wid = lax.axis_index("s") * _NC + lax.axis_index("c")
        iota = lax.iota(jnp.int32, 16)
        rows_t = [16 * g + iota for g in range(2)]

        def pair_body(k, carry):
            p = wid + k * _NW
            lb = p // _BB
            bb = p % _BB
            pltpu.sync_copy(xt_hbm.at[lb, bb], idx_v)

            def li_body(q, carry2):
                pltpu.async_copy(tbl_hbm.at[idx_v.at[q]], rows_v, sem).wait()
                for b in range(128):
                    for g in range(2):
                        v = rows_v[b, pl.ds(g * 16, 16)]
                        plsc.store_scatter(
                            outt_v, [rows_t[g], iota * 0 + b], v
                        )
                row0 = ((lb * 8 + q) * 4) * _BB + bb
                for db in range(4):
                    pltpu.sync_copy(
                        outt_v.at[pl.ds(8 * db, 8), :],
                        o_hbm.at[row0 + db * _BB],
                    )
                return carry2

            lax.fori_loop(0, 8, li_body, 0)
            return carry

        lax.fori_loop(0, _PER_W, pair_body, 0)

    return gat


_detile = _make_detile()
_gather = _make_gather()


def kernel(x, weight):
    wt = weight.T
    tbl4 = _detile(wt)
    tbl = tbl4.reshape(VOCAB, D)
    xq = x.T.reshape(_LB, 8, _BB, 128).transpose(0, 2, 1, 3)
    t = _gather(tbl, xq)
    t5 = t.reshape(L, D // 8, _BB, 8, 128)
    return t5.transpose(2, 4, 0, 1, 3).reshape(B, L, D)


# speedup vs baseline: 11.8681x; 2.7712x over previous
import functools

import jax
import jax.numpy as jnp
from jax import lax
from jax.experimental import pallas as pl
from jax.experimental.pallas import tpu as pltpu
from jax.experimental.pallas import tpu_sc as plsc

VOCAB = 1000000
D = 32
B = 16384
L = 200

_NC = 2
_NS = 16
_NW = _NC * _NS
_VTILES = VOCAB // 128
_NT = _VTILES + 1
_TAIL_W = _VTILES % _NW
_LB = L // 8
_BB = B // 128
_PAIRS = _LB * _BB
_PER_W = _PAIRS // _NW
_NTILES_OUT = L * (D // 8) * _BB

_MESH = dict(core_axis_name="c", subcore_axis_name="s")


def _make_detile():
    @functools.partial(
        pl.kernel,
        mesh=plsc.VectorSubcoreMesh(**_MESH),
        compiler_params=pltpu.CompilerParams(
            use_tc_tiling_on_sc=True, needs_layout_passes=False
        ),
        out_type=jax.ShapeDtypeStruct((VOCAB * D // 128, 128), jnp.float32),
        scratch_types=[
            pltpu.VMEM((2, D, 128), jnp.float32),
            pltpu.VMEM((2, D, 128), jnp.float32),
            pltpu.SemaphoreType.DMA,
            pltpu.SemaphoreType.DMA,
        ],
    )
    def detile(wt_hbm, out2_hbm, slab2, outb2, sem_l, sem_w):
        wid = lax.axis_index("s") * _NC + lax.axis_index("c")
        iota = lax.iota(jnp.int32, 16)
        rows_g = [(16 * g + iota) // 4 for g in range(8)]
        cols_g = [((16 * g + iota) % 4) * D for g in range(8)]
        nsteps = (_NT + _NW - 1) // _NW

        pltpu.async_copy(
            wt_hbm.at[:, pl.ds(wid * 128, 128)], slab2.at[0], sem_l
        )

        def tile_body(j, carry):
            t = wid + j * _NW

            @pl.when(t < _NT)
            def _():
                @pl.when(t + _NW < _NT)
                def _():
                    pltpu.async_copy(
                        wt_hbm.at[:, pl.ds((t + _NW) * 128, 128)],
                        slab2.at[(j + 1) % 2],
                        sem_l,
                    )

                pltpu.make_async_copy(
                    wt_hbm.at[:, pl.ds(0, 128)], slab2.at[0], sem_l
                ).wait()

                @pl.when(j >= 2)
                def _():
                    pltpu.make_async_copy(
                        wt_hbm.at[:, pl.ds(0, 128)], outb2.at[0], sem_w
                    ).wait()

                for d in range(D):
                    for g in range(8):
                        v = slab2[j % 2, d, pl.ds(g * 16, 16)]
                        plsc.store_scatter(
                            outb2.at[j % 2], [rows_g[g], cols_g[g] + d], v
                        )

                @pl.when(t < _VTILES)
                def _():
                    pltpu.async_copy(
                        outb2.at[j % 2],
                        out2_hbm.at[pl.ds(t * 32, 32), :],
                        sem_w,
                    )

                @pl.when(t == _VTILES)
                def _():
                    pltpu.async_copy(
                        outb2.at[j % 2, pl.ds(0, 16), :],
                        out2_hbm.at[pl.ds(t * 32, 16), :],
                        sem_w,
                    )

            return carry

        lax.fori_loop(0, nsteps, tile_body, 0)

        pltpu.make_async_copy(
            wt_hbm.at[:, pl.ds(0, 128)], outb2.at[0], sem_w
        ).wait()

        @pl.when(wid != _TAIL_W)
        def _():
            pltpu.make_async_copy(
                wt_hbm.at[:, pl.ds(0, 128)], outb2.at[0], sem_w
            ).wait()

        @pl.when(wid == _TAIL_W)
        def _():
            pltpu.make_async_copy(
                wt_hbm.at[:, pl.ds(0, 128)], outb2.at[0, pl.ds(0, 16), :],
                sem_w,
            ).wait()

    return detile


def _make_gather():
    @functools.partial(
        pl.kernel,
        mesh=plsc.VectorSubcoreMesh(**_MESH),
        compiler_params=pltpu.CompilerParams(
            use_tc_tiling_on_sc=False, needs_layout_passes=False
        ),
        out_type=jax.ShapeDtypeStruct((_NTILES_OUT, 8, 128), jnp.float32),
        scratch_types=[
            pltpu.VMEM((2, 8, 128), jnp.int32),
            pltpu.VMEM((2, 128, D), jnp.float32),
            pltpu.VMEM((2, D, 129), jnp.float32),
            pltpu.SemaphoreType.DMA,
            pltpu.SemaphoreType.DMA,
            pltpu.SemaphoreType.DMA,
        ],
    )
    def gat(tbl_hbm, xt_hbm, o_hbm, idx2, rows2, outt2, sem_i, sem_g, sem_w):
        wid = lax.axis_index("s") * _NC + lax.axis_index("c")
        iota = lax.iota(jnp.int32, 16)
        rows_t = [16 * g + iota for g in range(2)]
        total = _PER_W * 8

        pltpu.sync_copy(xt_hbm.at[wid // _BB, wid % _BB], idx2.at[0])
        pltpu.async_copy(tbl_hbm.at[idx2.at[0, 0]], rows2.at[0], sem_g)

        def step(s, carry):
            k = s // 8
            q = s % 8
            p = wid + k * _NW
            lb = p // _BB
            bb = p % _BB

            @pl.when((q == 0) & (k + 1 < _PER_W))
            def _():
                pn = wid + (k + 1) * _NW
                pltpu.async_copy(
                    xt_hbm.at[pn // _BB, pn % _BB],
                    idx2.at[(k + 1) % 2],
                    sem_i,
                )

            @pl.when(s + 1 < total)
            def _():
                sn = s + 1
                kn = sn // 8
                qn = sn % 8

                @pl.when(qn == 0)
                def _():
                    pltpu.make_async_copy(
                        o_hbm.at[0], idx2.at[0], sem_i
                    ).wait()

                pltpu.async_copy(
                    tbl_hbm.at[idx2.at[kn % 2, qn]], rows2.at[sn % 2], sem_g
                )

            pltpu.make_async_copy(o_hbm.at[0], rows2.at[0], sem_g).wait()

            @pl.when(s >= 2)
            def _():
                pltpu.make_async_copy(o_hbm.at[0], rows2.at[0], sem_w).wait()

            for b in range(128):
                for g in range(2):
                    v = rows2[s % 2, b, pl.ds(g * 16, 16)]
                    plsc.store_scatter(
                        outt2.at[s % 2], [rows_t[g], iota * 0 + b], v
                    )

            row0 = (lb * 8 + q) * 4 * _BB + bb
            for db in range(4):
                pltpu.async_copy(
                    outt2.at[s % 2, pl.ds(8 * db, 8), pl.ds(0, 128)],
                    o_hbm.at[row0 + db * _BB],
                    sem_w,
                )
            return carry

        lax.fori_loop(0, total, step, 0)

        pltpu.make_async_copy(o_hbm.at[0], rows2.at[0], sem_w).wait()
        pltpu.make_async_copy(o_hbm.at[0], rows2.at[0], sem_w).wait()

    return gat


_detile = _make_detile()
_gather = _make_gather()


def kernel(x, weight):
    wt = weight.T
    tbl4 = _detile(wt)
    tbl = tbl4.reshape(VOCAB, D)
    xq = x.T.reshape(_LB, 8, _BB, 128).transpose(0, 2, 1, 3)
    t = _gather(tbl, xq)
    t5 = t.reshape(L, D // 8, _BB, 8, 128)
    return t5.transpose(2, 4, 0, 1, 3).reshape(B, L, D)


# speedup vs baseline: 13.2904x; 1.1199x over previous
import functools

import jax
import jax.numpy as jnp
from jax import lax
from jax.experimental import pallas as pl
from jax.experimental.pallas import tpu as pltpu
from jax.experimental.pallas import tpu_sc as plsc

VOCAB = 1000000
D = 32
B = 16384
L = 200

_NC = 2
_NS = 16
_NW = _NC * _NS
_VTILES = VOCAB // 128
_NT = _VTILES + 1
_TAIL_W = _VTILES % _NW
_LB = L // 8
_BB = B // 128
_PAIRS = _LB * _BB
_PER_W = _PAIRS // _NW
_NTILES_OUT = L * (D // 8) * _BB

_MESH = dict(core_axis_name="c", subcore_axis_name="s")


_TCBLKV = 2048


def _make_detile():
    def body(wt_ref, out_ref):
        blk = wt_ref[...]
        t = blk.T
        t3 = t.reshape(_TCBLKV // 4, 4, D)
        for q in range(4):
            out_ref[:, q * D:(q + 1) * D] = t3[:, q, :]

    return pl.pallas_call(
        body,
        grid=(pl.cdiv(VOCAB, _TCBLKV),),
        in_specs=[pl.BlockSpec((D, _TCBLKV), lambda j: (0, j))],
        out_specs=pl.BlockSpec((_TCBLKV // 4, 128), lambda j: (j, 0)),
        out_shape=jax.ShapeDtypeStruct((VOCAB * D // 128, 128), jnp.float32),
    )


def _make_gather():
    @functools.partial(
        pl.kernel,
        mesh=plsc.VectorSubcoreMesh(**_MESH),
        compiler_params=pltpu.CompilerParams(
            use_tc_tiling_on_sc=False, needs_layout_passes=False
        ),
        out_type=jax.ShapeDtypeStruct((_NTILES_OUT, 8, 128), jnp.float32),
        scratch_types=[
            pltpu.VMEM((2, 8, 128), jnp.int32),
            pltpu.VMEM((2, 128, D), jnp.float32),
            pltpu.VMEM((2, D, 129), jnp.float32),
            pltpu.SemaphoreType.DMA,
            pltpu.SemaphoreType.DMA,
            pltpu.SemaphoreType.DMA,
        ],
    )
    def gat(tbl_hbm, xt_hbm, o_hbm, idx2, rows2, outt2, sem_i, sem_g, sem_w):
        wid = lax.axis_index("s") * _NC + lax.axis_index("c")
        iota = lax.iota(jnp.int32, 16)
        rows_t = [16 * g + iota for g in range(2)]
        total = _PER_W * 8

        pltpu.sync_copy(xt_hbm.at[wid // _BB, wid % _BB], idx2.at[0])
        pltpu.async_copy(tbl_hbm.at[idx2.at[0, 0]], rows2.at[0], sem_g)

        def step(s, carry):
            k = s // 8
            q = s % 8
            p = wid + k * _NW
            lb = p // _BB
            bb = p % _BB

            @pl.when((q == 0) & (k + 1 < _PER_W))
            def _():
                pn = wid + (k + 1) * _NW
                pltpu.async_copy(
                    xt_hbm.at[pn // _BB, pn % _BB],
                    idx2.at[(k + 1) % 2],
                    sem_i,
                )

            @pl.when(s + 1 < total)
            def _():
                sn = s + 1
                kn = sn // 8
                qn = sn % 8

                @pl.when(qn == 0)
                def _():
                    pltpu.make_async_copy(
                        o_hbm.at[0], idx2.at[0], sem_i
                    ).wait()

                pltpu.async_copy(
                    tbl_hbm.at[idx2.at[kn % 2, qn]], rows2.at[sn % 2], sem_g
                )

            pltpu.make_async_copy(o_hbm.at[0], rows2.at[0], sem_g).wait()

            @pl.when(s >= 2)
            def _():
                pltpu.make_async_copy(o_hbm.at[0], rows2.at[0], sem_w).wait()

            for b in range(128):
                for g in range(2):
                    v = rows2[s % 2, b, pl.ds(g * 16, 16)]
                    plsc.store_scatter(
                        outt2.at[s % 2], [rows_t[g], iota * 0 + b], v
                    )

            row0 = (lb * 8 + q) * 4 * _BB + bb
            for db in range(4):
                pltpu.async_copy(
                    outt2.at[s % 2, pl.ds(8 * db, 8), pl.ds(0, 128)],
                    o_hbm.at[row0 + db * _BB],
                    sem_w,
                )
            return carry

        lax.fori_loop(0, total, step, 0)

        pltpu.make_async_copy(o_hbm.at[0], rows2.at[0], sem_w).wait()
        pltpu.make_async_copy(o_hbm.at[0], rows2.at[0], sem_w).wait()

    return gat


_detile = _make_detile()
_gather = _make_gather()


def kernel(x, weight):
    wt = weight.T
    tbl4 = _detile(wt)
    tbl = tbl4.reshape(VOCAB, D)
    xq = x.T.reshape(_LB, 8, _BB, 128).transpose(0, 2, 1, 3)
    t = _gather(tbl, xq)
    t5 = t.reshape(L, D // 8, _BB, 8, 128)
    return t5.transpose(2, 4, 0, 1, 3).reshape(B, L, D)


# speedup vs baseline: 14.4870x; 1.0900x over previous
import functools

import jax
import jax.numpy as jnp
from jax import lax
from jax.experimental import pallas as pl
from jax.experimental.pallas import tpu as pltpu
from jax.experimental.pallas import tpu_sc as plsc

VOCAB = 1000000
D = 32
B = 16384
L = 200

_NC = 2
_NS = 16
_NW = _NC * _NS
_VTILES = VOCAB // 128
_NT = _VTILES + 1
_TAIL_W = _VTILES % _NW
_LB = L // 8
_BB = B // 128
_PAIRS = _LB * _BB
_PER_W = _PAIRS // _NW
_NTILES_OUT = L * (D // 8) * _BB

_MESH = dict(core_axis_name="c", subcore_axis_name="s")


_TCBLKV = 8192


def _make_detile():
    def body(wt_ref, out_ref):
        blk = wt_ref[...]
        i = lax.broadcasted_iota(jnp.int32, (D, D), 0)
        j = lax.broadcasted_iota(jnp.int32, (D, D), 1)
        ident = (i == j).astype(jnp.float32)
        t = lax.dot_general(
            blk, ident, (((0,), (0,)), ((), ())),
            preferred_element_type=jnp.float32,
        )
        t3 = t.reshape(_TCBLKV // 4, 4, D)
        for q in range(4):
            out_ref[:, q * D:(q + 1) * D] = t3[:, q, :]

    return pl.pallas_call(
        body,
        grid=(pl.cdiv(VOCAB, _TCBLKV),),
        in_specs=[pl.BlockSpec((D, _TCBLKV), lambda j: (0, j))],
        out_specs=pl.BlockSpec((_TCBLKV // 4, 128), lambda j: (j, 0)),
        out_shape=jax.ShapeDtypeStruct((VOCAB * D // 128, 128), jnp.float32),
    )


def _make_gather():
    @functools.partial(
        pl.kernel,
        mesh=plsc.VectorSubcoreMesh(**_MESH),
        compiler_params=pltpu.CompilerParams(
            use_tc_tiling_on_sc=False, needs_layout_passes=False
        ),
        out_type=jax.ShapeDtypeStruct((_NTILES_OUT, 8, 128), jnp.float32),
        scratch_types=[
            pltpu.VMEM((2, 8, 128), jnp.int32),
            pltpu.VMEM((4, 128, D), jnp.float32),
            pltpu.VMEM((2, D, 129), jnp.float32),
            pltpu.SemaphoreType.DMA,
            pltpu.SemaphoreType.DMA,
            pltpu.SemaphoreType.DMA,
        ],
    )
    def gat(tbl_hbm, xt_hbm, o_hbm, idx2, rows2, outt2, sem_i, sem_g, sem_w):
        wid = lax.axis_index("s") * _NC + lax.axis_index("c")
        iota = lax.iota(jnp.int32, 16)
        rows_t = [16 * g + iota for g in range(2)]
        total = _PER_W * 8
        _LOOKAHEAD = 3

        pltpu.sync_copy(xt_hbm.at[wid // _BB, wid % _BB], idx2.at[0])
        for s0 in range(_LOOKAHEAD):
            pltpu.async_copy(
                tbl_hbm.at[idx2.at[0, s0]], rows2.at[s0], sem_g
            )

        def step(s, carry):
            k = s // 8
            q = s % 8
            p = wid + k * _NW
            lb = p // _BB
            bb = p % _BB

            @pl.when((q == 0) & (k + 1 < _PER_W))
            def _():
                pn = wid + (k + 1) * _NW
                pltpu.async_copy(
                    xt_hbm.at[pn // _BB, pn % _BB],
                    idx2.at[(k + 1) % 2],
                    sem_i,
                )

            @pl.when(s + _LOOKAHEAD < total)
            def _():
                sn = s + _LOOKAHEAD
                kn = sn // 8
                qn = sn % 8

                @pl.when(qn == 0)
                def _():
                    pltpu.make_async_copy(
                        o_hbm.at[0], idx2.at[0], sem_i
                    ).wait()

                pltpu.async_copy(
                    tbl_hbm.at[idx2.at[kn % 2, qn]], rows2.at[sn % 4], sem_g
                )

            pltpu.make_async_copy(o_hbm.at[0], rows2.at[0], sem_g).wait()

            @pl.when(s >= 2)
            def _():
                pltpu.make_async_copy(o_hbm.at[0], rows2.at[0], sem_w).wait()

            for b in range(128):
                for g in range(2):
                    v = rows2[s % 4, b, pl.ds(g * 16, 16)]
                    plsc.store_scatter(
                        outt2.at[s % 2], [rows_t[g], iota * 0 + b], v
                    )

            row0 = (lb * 8 + q) * 4 * _BB + bb
            for db in range(4):
                pltpu.async_copy(
                    outt2.at[s % 2, pl.ds(8 * db, 8), pl.ds(0, 128)],
                    o_hbm.at[row0 + db * _BB],
                    sem_w,
                )
            return carry

        lax.fori_loop(0, total, step, 0)

        pltpu.make_async_copy(o_hbm.at[0], rows2.at[0], sem_w).wait()
        pltpu.make_async_copy(o_hbm.at[0], rows2.at[0], sem_w).wait()

    return gat


_detile = _make_detile()
_gather = _make_gather()


def kernel(x, weight):
    wt = weight.T
    tbl4 = _detile(wt)
    tbl = tbl4.reshape(VOCAB, D)
    xq = x.T.reshape(_LB, 8, _BB, 128).transpose(0, 2, 1, 3)
    t = _gather(tbl, xq)
    t5 = t.reshape(L, D // 8, _BB, 8, 128)
    return t5.transpose(2, 4, 0, 1, 3).reshape(B, L, D)


# speedup vs baseline: 14.7847x; 1.0205x over previous
import functools

import jax
import jax.numpy as jnp
from jax import lax
from jax.experimental import pallas as pl
from jax.experimental.pallas import tpu as pltpu
from jax.experimental.pallas import tpu_sc as plsc

VOCAB = 1000000
D = 32
B = 16384
L = 200

_NC = 2
_NS = 16
_NW = _NC * _NS
_VTILES = VOCAB // 128
_NT = _VTILES + 1
_TAIL_W = _VTILES % _NW
_LB = L // 8
_BB = B // 128
_PAIRS = _LB * _BB
_PER_W = _PAIRS // _NW
_NTILES_OUT = L * (D // 8) * _BB

_MESH = dict(core_axis_name="c", subcore_axis_name="s")


_TCBLKV = 8192


def _make_detile():
    def body(wt_ref, out_ref):
        blk = wt_ref[...]
        t = blk.T
        t3 = t.reshape(_TCBLKV // 4, 4, D)
        for q in range(4):
            out_ref[:, q * D:(q + 1) * D] = t3[:, q, :]

    return pl.pallas_call(
        body,
        grid=(pl.cdiv(VOCAB, _TCBLKV),),
        in_specs=[pl.BlockSpec((D, _TCBLKV), lambda j: (0, j))],
        out_specs=pl.BlockSpec((_TCBLKV // 4, 128), lambda j: (j, 0)),
        out_shape=jax.ShapeDtypeStruct((VOCAB * D // 128, 128), jnp.float32),
    )


def _make_gather():
    @functools.partial(
        pl.kernel,
        mesh=plsc.VectorSubcoreMesh(**_MESH),
        compiler_params=pltpu.CompilerParams(
            use_tc_tiling_on_sc=False, needs_layout_passes=False
        ),
        out_type=jax.ShapeDtypeStruct((_NTILES_OUT, 8, 128), jnp.float32),
        scratch_types=[
            pltpu.VMEM((2, 8, 128), jnp.int32),
            pltpu.VMEM((4, 128, D), jnp.float32),
            pltpu.VMEM((2, D, 129), jnp.float32),
            pltpu.SemaphoreType.DMA,
            pltpu.SemaphoreType.DMA,
            pltpu.SemaphoreType.DMA,
        ],
    )
    def gat(tbl_hbm, xt_hbm, o_hbm, idx2, rows2, outt2, sem_i, sem_g, sem_w):
        wid = lax.axis_index("s") * _NC + lax.axis_index("c")
        iota = lax.iota(jnp.int32, 16)
        rows_t = [16 * g + iota for g in range(2)]
        total = _PER_W * 8
        _LOOKAHEAD = 3

        pltpu.sync_copy(xt_hbm.at[wid // _BB, wid % _BB], idx2.at[0])
        for s0 in range(_LOOKAHEAD):
            pltpu.async_copy(
                tbl_hbm.at[idx2.at[0, s0]], rows2.at[s0], sem_g
            )

        def step(s, carry):
            k = s // 8
            q = s % 8
            p = wid + k * _NW
            lb = p // _BB
            bb = p % _BB

            @pl.when((q == 0) & (k + 1 < _PER_W))
            def _():
                pn = wid + (k + 1) * _NW
                pltpu.async_copy(
                    xt_hbm.at[pn // _BB, pn % _BB],
                    idx2.at[(k + 1) % 2],
                    sem_i,
                )

            @pl.when(s + _LOOKAHEAD < total)
            def _():
                sn = s + _LOOKAHEAD
                kn = sn // 8
                qn = sn % 8

                @pl.when(qn == 0)
                def _():
                    pltpu.make_async_copy(
                        o_hbm.at[0], idx2.at[0], sem_i
                    ).wait()

                pltpu.async_copy(
                    tbl_hbm.at[idx2.at[kn % 2, qn]], rows2.at[sn % 4], sem_g
                )

            pltpu.make_async_copy(o_hbm.at[0], rows2.at[0], sem_g).wait()

            @pl.when(s >= 2)
            def _():
                pltpu.make_async_copy(o_hbm.at[0], rows2.at[0], sem_w).wait()

            for b in range(128):
                for g in range(2):
                    v = rows2[s % 4, b, pl.ds(g * 16, 16)]
                    plsc.store_scatter(
                        outt2.at[s % 2], [rows_t[g], iota * 0 + b], v
                    )

            row0 = (lb * 8 + q) * 4 * _BB + bb
            for db in range(4):
                pltpu.async_copy(
                    outt2.at[s % 2, pl.ds(8 * db, 8), pl.ds(0, 128)],
                    o_hbm.at[row0 + db * _BB],
                    sem_w,
                )
            return carry

        lax.fori_loop(0, total, step, 0)

        pltpu.make_async_copy(o_hbm.at[0], rows2.at[0], sem_w).wait()
        pltpu.make_async_copy(o_hbm.at[0], rows2.at[0], sem_w).wait()

    return gat


_detile = _make_detile()
_gather = _make_gather()


def kernel(x, weight):
    wt = weight.T
    tbl4 = _detile(wt)
    tbl = tbl4.reshape(VOCAB, D)
    xq = x.T.reshape(_LB, 8, _BB, 128).transpose(0, 2, 1, 3)
    t = _gather(tbl, xq)
    t5 = t.reshape(L, D // 8, _BB, 8, 128)
    return t5.transpose(2, 4, 0, 1, 3).reshape(B, L, D)
